# baseline (speedup 1.0000x reference)
import functools

import jax
import jax.numpy as jnp
from jax import lax
from jax.experimental import pallas as pl
from jax.experimental.pallas import tpu as pltpu
from jax.experimental.pallas import tpu_sc as plsc

B = 16384
N_NUM = 13
N_CAT = 26
N_TOK = N_NUM + N_CAT
CARD = 100000
D = 32

NC = 2
NS = 16
NW = NC * NS
S_PER_W = B // NW
C = 64
NCHUNK = S_PER_W // C
R = C * N_CAT
GRP = R // 128

_mesh = plsc.VectorSubcoreMesh(core_axis_name="c", subcore_axis_name="s")


@functools.partial(
    pl.kernel,
    out_type=jax.ShapeDtypeStruct((B * N_TOK, D), jnp.float32),
    mesh=_mesh,
    compiler_params=pltpu.CompilerParams(use_tc_tiling_on_sc=False),
    scratch_types=[
        pltpu.VMEM((R,), jnp.int32),
        pltpu.VMEM((R,), jnp.int32),
        pltpu.VMEM((R,), jnp.int32),
        pltpu.VMEM((GRP, 128), jnp.int32),
        pltpu.VMEM((GRP, 128), jnp.int32),
        pltpu.VMEM((R, D), jnp.float32),
        pltpu.SemaphoreType.DMA,
        pltpu.SemaphoreType.DMA,
    ],
)
def _cat_gather(xcat_hbm, table_hbm, out_hbm, xc_v, foff_v, obase_v, idx_v,
                oidx_v, rows_v, gsem, ssem):
    wid = lax.axis_index("s") * NC + lax.axis_index("c")

    def pre(g, carry):
        p = g * 16 + lax.iota(jnp.int32, 16)
        fld = p % N_CAT
        foff_v[pl.ds(g * 16, 16)] = fld * CARD
        obase_v[pl.ds(g * 16, 16)] = (
            p + lax.shift_right_logical(p - fld, 1) + N_NUM)
        return carry

    lax.fori_loop(0, R // 16, pre, None)

    def chunk(k, carry):
        b0 = wid * S_PER_W + k * C
        pltpu.sync_copy(xcat_hbm.at[pl.ds(b0 * N_CAT, R)], xc_v)
        ob_add = b0 * N_TOK

        def idxcalc(g, c2):
            r = g // 8
            col = (g % 8) * 16
            xv = xc_v[pl.ds(g * 16, 16)]
            idx_v[r, pl.ds(col, 16)] = (
                jnp.maximum(xv, 0) + foff_v[pl.ds(g * 16, 16)])
            oidx_v[r, pl.ds(col, 16)] = obase_v[pl.ds(g * 16, 16)] + ob_add
            return c2

        lax.fori_loop(0, R // 16, idxcalc, None)

        gathers = [
            pltpu.async_copy(table_hbm.at[idx_v.at[g]],
                             rows_v.at[pl.ds(g * 128, 128)], gsem)
            for g in range(GRP)
        ]
        for cp in gathers:
            cp.wait()
        scatters = [
            pltpu.async_copy(rows_v.at[pl.ds(g * 128, 128)],
                             out_hbm.at[oidx_v.at[g]], ssem)
            for g in range(GRP)
        ]
        for cp in scatters:
            cp.wait()
        return carry

    lax.fori_loop(0, NCHUNK, chunk, None)


BB = 512


def _num_body(x_ref, w_ref, b_ref, buf_ref, o_ref, acc_ref, sem):
    del buf_ref
    i = pl.program_id(0)
    acc_ref[...] = (x_ref[...][:, :, None] * w_ref[...][None, :, :]
                    + b_ref[...][None, :, :])
    cp = pltpu.make_async_copy(
        acc_ref, o_ref.at[pl.ds(i * BB, BB), pl.ds(0, N_NUM), :], sem)
    cp.start()
    cp.wait()


def _num_fill(x_num, num_weight, num_bias, buf3):
    return pl.pallas_call(
        _num_body,
        grid=(B // BB,),
        in_specs=[
            pl.BlockSpec((BB, N_NUM), lambda i: (i, 0)),
            pl.BlockSpec((N_NUM, D), lambda i: (0, 0)),
            pl.BlockSpec((N_NUM, D), lambda i: (0, 0)),
            pl.BlockSpec(memory_space=pltpu.MemorySpace.HBM),
        ],
        out_specs=pl.BlockSpec(memory_space=pltpu.MemorySpace.HBM),
        out_shape=jax.ShapeDtypeStruct((B, N_TOK, D), jnp.float32),
        scratch_shapes=[
            pltpu.VMEM((BB, N_NUM, D), jnp.float32),
            pltpu.SemaphoreType.DMA,
        ],
        input_output_aliases={3: 0},
    )(x_num, num_weight, num_bias, buf3)


def kernel(x_num, x_cat, num_weight, num_bias, cat_table):
    xc = x_cat.astype(jnp.int32).reshape(B * N_CAT)
    flat = _cat_gather(xc, cat_table)
    buf3 = flat.reshape(B, N_TOK, D)
    return _num_fill(x_num, num_weight, num_bias, buf3)


# speedup vs baseline: 1.1633x; 1.1633x over previous
import functools

import jax
import jax.numpy as jnp
from jax import lax
from jax.experimental import pallas as pl
from jax.experimental.pallas import tpu as pltpu
from jax.experimental.pallas import tpu_sc as plsc

B = 16384
N_NUM = 13
N_CAT = 26
N_TOK = N_NUM + N_CAT
CARD = 100000
D = 32

NC = 2
NS = 16
NW = NC * NS
S_PER_W = B // NW

_mesh = plsc.VectorSubcoreMesh(core_axis_name="c", subcore_axis_name="s")


@functools.partial(
    pl.kernel,
    out_type=jax.ShapeDtypeStruct((N_CAT * B, D), jnp.float32),
    mesh=_mesh,
    compiler_params=pltpu.CompilerParams(use_tc_tiling_on_sc=False),
    scratch_types=[
        pltpu.VMEM((S_PER_W,), jnp.int32),
        pltpu.VMEM((4, 128), jnp.int32),
        pltpu.VMEM((S_PER_W, D), jnp.float32),
        pltpu.SemaphoreType.DMA,
    ],
)
def _cat_gather(xcat_hbm, table_hbm, out_hbm, xc_v, idx_v, rows_v, sem):
    wid = lax.axis_index("s") * NC + lax.axis_index("c")
    b0 = wid * S_PER_W

    def per_field(j, carry):
        pltpu.sync_copy(xcat_hbm.at[pl.ds(j * B + b0, S_PER_W)], xc_v)
        off = j * CARD

        def grp(g, c2):
            xv = xc_v[pl.ds(g * 16, 16)]
            idx_v[g // 8, pl.ds((g % 8) * 16, 16)] = jnp.maximum(xv, 0) + off
            return c2

        lax.fori_loop(0, S_PER_W // 16, grp, None)

        gathers = [
            pltpu.async_copy(table_hbm.at[idx_v.at[q]],
                             rows_v.at[pl.ds(q * 128, 128)], sem)
            for q in range(S_PER_W // 128)
        ]
        for cp in gathers:
            cp.wait()
        pltpu.sync_copy(rows_v, out_hbm.at[pl.ds(j * B + b0, S_PER_W)])
        return carry

    lax.fori_loop(0, N_CAT, per_field, None)


BL = 512


def _asm_body(x_ref, w_ref, b_ref, c_ref, o_ref):
    o_ref[0:N_NUM] = (x_ref[...][:, None, :] * w_ref[...][:, :, None]
                      + b_ref[...][:, :, None])
    o_ref[N_NUM:N_TOK] = jnp.transpose(c_ref[...], (0, 2, 1))


def _tc_assemble(xnT, num_weight, num_bias, catc3):
    return pl.pallas_call(
        _asm_body,
        grid=(B // BL,),
        in_specs=[
            pl.BlockSpec((N_NUM, BL), lambda i: (0, i)),
            pl.BlockSpec((N_NUM, D), lambda i: (0, 0)),
            pl.BlockSpec((N_NUM, D), lambda i: (0, 0)),
            pl.BlockSpec((N_CAT, BL, D), lambda i: (0, i, 0)),
        ],
        out_specs=pl.BlockSpec((N_TOK, D, BL), lambda i: (0, 0, i)),
        out_shape=jax.ShapeDtypeStruct((N_TOK, D, B), jnp.float32),
    )(xnT, num_weight, num_bias, catc3)


def kernel(x_num, x_cat, num_weight, num_bias, cat_table):
    xcT = jnp.transpose(x_cat.astype(jnp.int32), (1, 0)).reshape(N_CAT * B)
    xnT = jnp.transpose(x_num, (1, 0))
    catc = _cat_gather(xcT, cat_table)
    catc3 = catc.reshape(N_CAT, B, D)
    out3 = _tc_assemble(xnT, num_weight, num_bias, catc3)
    return jnp.transpose(out3, (2, 0, 1))


# speedup vs baseline: 1.2004x; 1.0319x over previous
import functools

import jax
import jax.numpy as jnp
from jax import lax
from jax.experimental import pallas as pl
from jax.experimental.pallas import tpu as pltpu
from jax.experimental.pallas import tpu_sc as plsc

B = 16384
N_NUM = 13
N_CAT = 26
N_TOK = N_NUM + N_CAT
CARD = 100000
D = 32

NC = 2
NS = 16
NW = NC * NS
S_PER_W = B // NW

_mesh = plsc.VectorSubcoreMesh(core_axis_name="c", subcore_axis_name="s")


@functools.partial(
    pl.kernel,
    out_type=jax.ShapeDtypeStruct((N_CAT * B, D), jnp.float32),
    mesh=_mesh,
    compiler_params=pltpu.CompilerParams(use_tc_tiling_on_sc=False),
    scratch_types=[
        pltpu.VMEM((S_PER_W,), jnp.int32),
        pltpu.VMEM((4, 128), jnp.int32),
        pltpu.VMEM((S_PER_W, D), jnp.float32),
        pltpu.SemaphoreType.DMA,
    ],
)
def _cat_gather(xcat_hbm, table_hbm, out_hbm, xc_v, idx_v, rows_v, sem):
    wid = lax.axis_index("s") * NC + lax.axis_index("c")
    b0 = wid * S_PER_W

    def per_field(j, carry):
        pltpu.sync_copy(xcat_hbm.at[pl.ds(j * B + b0, S_PER_W)], xc_v)
        off = j * CARD

        def grp(g, c2):
            xv = xc_v[pl.ds(g * 16, 16)]
            idx_v[g // 8, pl.ds((g % 8) * 16, 16)] = (
                (jnp.maximum(xv, 0) + off) * 4)
            return c2

        lax.fori_loop(0, S_PER_W // 16, grp, None)

        gathers = [
            pltpu.async_copy(table_hbm.at[idx_v.at[q]],
                             rows_v.at[pl.ds(q * 128, 128)], sem)
            for q in range(S_PER_W // 128)
        ]
        for cp in gathers:
            cp.wait()
        pltpu.sync_copy(rows_v, out_hbm.at[pl.ds(j * B + b0, S_PER_W)])
        return carry

    lax.fori_loop(0, N_CAT, per_field, None)


TROWS = N_CAT * CARD
CH = 2048


def _conv_body(t_ref, o_ref):
    tt = jnp.transpose(t_ref[...], (1, 0))
    o_ref[...] = tt.reshape(CH // 4, 4 * D)


def _tc_relayout_table(tableT):
    return pl.pallas_call(
        _conv_body,
        grid=(pl.cdiv(TROWS, CH),),
        in_specs=[pl.BlockSpec((D, CH), lambda i: (0, i))],
        out_specs=pl.BlockSpec((CH // 4, 4 * D), lambda i: (i, 0)),
        out_shape=jax.ShapeDtypeStruct((TROWS // 4, 4 * D), jnp.float32),
    )(tableT)


BL = 512


def _asm_body(x_ref, w_ref, b_ref, c_ref, o_ref):
    o_ref[0:N_NUM] = (x_ref[...][:, None, :] * w_ref[...][:, :, None]
                      + b_ref[...][:, :, None])
    o_ref[N_NUM:N_TOK] = jnp.transpose(c_ref[...], (0, 2, 1))


def _tc_assemble(xnT, num_weight, num_bias, catc3):
    return pl.pallas_call(
        _asm_body,
        grid=(B // BL,),
        in_specs=[
            pl.BlockSpec((N_NUM, BL), lambda i: (0, i)),
            pl.BlockSpec((N_NUM, D), lambda i: (0, 0)),
            pl.BlockSpec((N_NUM, D), lambda i: (0, 0)),
            pl.BlockSpec((N_CAT, BL, D), lambda i: (0, i, 0)),
        ],
        out_specs=pl.BlockSpec((N_TOK, D, BL), lambda i: (0, 0, i)),
        out_shape=jax.ShapeDtypeStruct((N_TOK, D, B), jnp.float32),
    )(xnT, num_weight, num_bias, catc3)


def kernel(x_num, x_cat, num_weight, num_bias, cat_table):
    xcT = jnp.transpose(x_cat.astype(jnp.int32), (1, 0)).reshape(N_CAT * B)
    xnT = jnp.transpose(x_num, (1, 0))
    padded = jnp.pad(cat_table, ((0, 0), (0, 3 * D)))
    catc = _cat_gather(xcT, padded.reshape(4 * N_CAT * CARD, D))
    catc3 = catc.reshape(N_CAT, B, D)
    out3 = _tc_assemble(xnT, num_weight, num_bias, catc3)
    return jnp.transpose(out3, (2, 0, 1))
